# baseline (speedup 1.0000x reference)
import functools

import jax
import jax.numpy as jnp
from jax import lax
from jax.experimental import pallas as pl
from jax.experimental.pallas import tpu as pltpu
from jax.experimental.pallas import tpu_sc as plsc

N = 10000
DEG = 16
U = 256
RELS = 8
ENTS = 4
EDGES = N * DEG

NBLK = 1000
SBLK = 200
NEG = -1000000000.0

_NW = 32
NPAD = 10240
EPAD = NPAD * DEG
_CH = 64
_CNODES = _CH // DEG
_NCHP = EPAD // _CH // 16
_NCH0 = 112
_NCH1 = 48

def _projtab_body(ns_ref, pe_ref, adj_ref, rel_ref, pw_ref, qw_ref, kw_ref,
                  vw_ref, relw_ref, nso_ref, qso_ref, lin_ref, kt_ref,
                  vt_ref, nsb, kws, vws):
    r = pl.program_id(1)

    @pl.when(jnp.logical_and(pl.program_id(0) == 0, r == 0))
    def _():
        for rr in range(RELS):
            kws[rr] = jnp.dot(relw_ref[rr], kw_ref[...],
                              preferred_element_type=jnp.float32
                              ).astype(jnp.bfloat16)
            vws[rr] = jnp.dot(relw_ref[rr], vw_ref[...],
                              preferred_element_type=jnp.float32
                              ).astype(jnp.bfloat16)

    @pl.when(r == 0)
    def _():
        x = ns_ref[...]
        pe = pe_ref[...]
        acc = jnp.zeros((NBLK, U), jnp.float32)
        for e in range(ENTS):
            pm = jnp.where(pe == e, 1.0, 0.0)
            acc = acc + pm * jnp.dot(x, pw_ref[e],
                                     preferred_element_type=jnp.float32)
        nsb[...] = acc.astype(jnp.bfloat16)
        nso_ref[...] = acc
        qso_ref[...] = jnp.dot(acc, qw_ref[...],
                               preferred_element_type=jnp.float32) * (1.0 / U)
        lin_ref[...] = rel_ref[...] * N + jnp.maximum(adj_ref[...] - 1, 0)

    x = nsb[...]
    kt_ref[0] = jnp.dot(x, kws[r], preferred_element_type=jnp.float32)
    vt_ref[0] = jnp.dot(x, vws[r], preferred_element_type=jnp.float32)


def _projtab(node_state, point_enc, adjacency, relation_enc, pw, qw, kw, vw,
             relw):
    grid = (N // NBLK, RELS)
    return pl.pallas_call(
        _projtab_body,
        grid=grid,
        in_specs=[
            pl.BlockSpec((NBLK, U), lambda i, r: (i, 0)),
            pl.BlockSpec((NBLK, 1), lambda i, r: (i, 0)),
            pl.BlockSpec((NBLK, DEG), lambda i, r: (i, 0)),
            pl.BlockSpec((NBLK, DEG), lambda i, r: (i, 0)),
            pl.BlockSpec((ENTS, U, U), lambda i, r: (0, 0, 0)),
            pl.BlockSpec((U, U), lambda i, r: (0, 0)),
            pl.BlockSpec((U, U), lambda i, r: (0, 0)),
            pl.BlockSpec((U, U), lambda i, r: (0, 0)),
            pl.BlockSpec((RELS, U, U), lambda i, r: (0, 0, 0)),
        ],
        out_specs=[
            pl.BlockSpec((NBLK, U), lambda i, r: (i, 0)),
            pl.BlockSpec((NBLK, U), lambda i, r: (i, 0)),
            pl.BlockSpec((NBLK, DEG), lambda i, r: (i, 0)),
            pl.BlockSpec((1, NBLK, U), lambda i, r: (r, i, 0)),
            pl.BlockSpec((1, NBLK, U), lambda i, r: (r, i, 0)),
        ],
        out_shape=[
            jax.ShapeDtypeStruct((N, U), jnp.float32),
            jax.ShapeDtypeStruct((N, U), jnp.float32),
            jax.ShapeDtypeStruct((N, DEG), jnp.int32),
            jax.ShapeDtypeStruct((RELS, N, U), jnp.float32),
            jax.ShapeDtypeStruct((RELS, N, U), jnp.float32),
        ],
        scratch_shapes=[
            pltpu.VMEM((NBLK, U), jnp.bfloat16),
            pltpu.VMEM((RELS, U, U), jnp.bfloat16),
            pltpu.VMEM((RELS, U, U), jnp.bfloat16),
        ],
    )(node_state, point_enc, adjacency, relation_enc, pw, qw, kw, vw, relw)


def _perm16(x, idx):
    return lax.gather(
        x, idx[:, None],
        lax.GatherDimensionNumbers(offset_dims=(), collapsed_slice_dims=(0,),
                                   start_index_map=(0,)),
        slice_sizes=(1,),
        mode=lax.GatherScatterMode.PROMISE_IN_BOUNDS)


def _lanesum(x):
    lanes = lax.iota(jnp.int32, 16)
    for sh in (1, 2, 4, 8):
        x = x + _perm16(x, lanes ^ sh)
    return x


def _sc_scores(lin_pad, qs_pad, kt_flat):
    mesh = plsc.VectorSubcoreMesh(core_axis_name="c", subcore_axis_name="s")

    @functools.partial(
        pl.kernel,
        mesh=mesh,
        out_type=jax.ShapeDtypeStruct((NPAD, DEG), jnp.float32),
        scratch_types=[
            pltpu.VMEM((_NCH0 * _CH,), jnp.int32),
            pltpu.VMEM((_NCH0 * _CNODES, DEG), jnp.float32),
            pltpu.VMEM((_CH, U), jnp.float32),
            pltpu.VMEM((_CH, U), jnp.float32),
            pltpu.VMEM((_CNODES, U), jnp.float32),
            pltpu.VMEM((_CNODES, U), jnp.float32),
            pltpu.SemaphoreType.DMA,
            pltpu.SemaphoreType.DMA,
            pltpu.SemaphoreType.DMA,
            pltpu.SemaphoreType.DMA,
        ],
    )
    def scores(lin_hbm, qs_hbm, kt_hbm, raw_hbm, idx_v, sbuf, buf0, buf1,
               qb0, qb1, sem0, sem1, semq0, semq1):
        cid = lax.axis_index("c")
        sid = lax.axis_index("s")
        cbase = sid * _NCHP + cid * _NCH0
        nch = jnp.where(cid == 0, _NCH0, _NCH1)
        ebase = cbase * _CH
        nbase = cbase * _CNODES

        @pl.when(cid == 0)
        def _():
            pltpu.sync_copy(lin_hbm.at[pl.ds(ebase, _NCH0 * _CH)], idx_v)

        @pl.when(cid != 0)
        def _():
            pltpu.sync_copy(lin_hbm.at[pl.ds(ebase, _NCH1 * _CH)],
                            idx_v.at[pl.ds(0, _NCH1 * _CH)])

        def src(c):
            return kt_hbm.at[idx_v.at[pl.ds(c * _CH, _CH)]]

        def qsrc(c):
            return qs_hbm.at[pl.ds(nbase + c * _CNODES, _CNODES)]

        def compute(c, buf, qbuf):
            for g in range(_CNODES):
                nl = c * _CNODES + g
                qv = tuple(qbuf[g, pl.ds(cc * 16, 16)] for cc in range(16))

                def edge_body(d, srow):
                    e = g * DEG + d
                    acc = qv[0] * buf[e, pl.ds(0, 16)]
                    for cc in range(1, 16):
                        acc = acc + qv[cc] * buf[e, pl.ds(cc * 16, 16)]
                    s = _lanesum(acc)
                    return jnp.where(
                        lax.iota(jnp.int32, 16) == d, s, srow)

                srow = lax.fori_loop(0, DEG, edge_body,
                                     jnp.zeros((16,), jnp.float32))
                sbuf[nl] = srow

        pltpu.async_copy(src(0), buf0, sem0)
        pltpu.async_copy(qsrc(0), qb0, semq0)
        pltpu.async_copy(src(1), buf1, sem1)
        pltpu.async_copy(qsrc(1), qb1, semq1)

        def do(c, buf, qbuf, sem, semq, more):
            pltpu.make_async_copy(src(c), buf, sem).wait()
            pltpu.make_async_copy(qsrc(c), qbuf, semq).wait()
            compute(c, buf, qbuf)

            @pl.when(more)
            def _():
                pltpu.async_copy(src(c + 2), buf, sem)
                pltpu.async_copy(qsrc(c + 2), qbuf, semq)

        def step(p, carry):
            do(p * 2, buf0, qb0, sem0, semq0, p * 2 + 2 < nch)
            do(p * 2 + 1, buf1, qb1, sem1, semq1, p * 2 + 3 < nch)
            return carry

        lax.fori_loop(0, nch // 2, step, 0)

        @pl.when(cid == 0)
        def _():
            pltpu.sync_copy(sbuf,
                            raw_hbm.at[pl.ds(nbase, _NCH0 * _CNODES)])

        @pl.when(cid != 0)
        def _():
            pltpu.sync_copy(sbuf.at[pl.ds(0, _NCH1 * _CNODES)],
                            raw_hbm.at[pl.ds(nbase, _NCH1 * _CNODES)])

    return scores(lin_pad, qs_pad, kt_flat)


def _softmax_body(raw_ref, rel_ref, adj_ref, w_ref):
    raw = raw_ref[...]
    raw = jnp.where(adj_ref[...] == 0, 0.0, raw)
    raw = jnp.where(rel_ref[...] == 0, NEG, raw)
    m = jnp.max(raw, axis=0, keepdims=True)
    e = jnp.exp(raw - m)
    s = jnp.sum(e, axis=0, keepdims=True)
    w = e / s
    w_ref[...] = jnp.where(adj_ref[...] == 0, 0.0, w)


def _softmax0(raw_pad, rel_pad, adj_pad):
    return pl.pallas_call(
        _softmax_body,
        out_shape=jax.ShapeDtypeStruct((NPAD, DEG), jnp.float32),
    )(raw_pad, rel_pad, adj_pad)


def _sc_agg(lin_pad, w_pad, vt_flat):
    mesh = plsc.VectorSubcoreMesh(core_axis_name="c", subcore_axis_name="s")

    @functools.partial(
        pl.kernel,
        mesh=mesh,
        out_type=jax.ShapeDtypeStruct((NPAD, U), jnp.float32),
        scratch_types=[
            pltpu.VMEM((_NCH0 * _CH,), jnp.int32),
            pltpu.VMEM((_NCH0 * _CNODES, DEG), jnp.float32),
            pltpu.VMEM((_CNODES, U), jnp.float32),
            pltpu.VMEM((_CNODES, U), jnp.float32),
            pltpu.VMEM((_CH, U), jnp.float32),
            pltpu.VMEM((_CH, U), jnp.float32),
            pltpu.SemaphoreType.DMA,
            pltpu.SemaphoreType.DMA,
            pltpu.SemaphoreType.DMA,
            pltpu.SemaphoreType.DMA,
        ],
    )
    def agg(lin_hbm, w_hbm, vt_hbm, agg_hbm, idx_v, wbuf, abuf0, abuf1,
            buf0, buf1, sem0, sem1, semw0, semw1):
        cid = lax.axis_index("c")
        sid = lax.axis_index("s")
        cbase = sid * _NCHP + cid * _NCH0
        nch = jnp.where(cid == 0, _NCH0, _NCH1)
        ebase = cbase * _CH
        nbase = cbase * _CNODES

        @pl.when(cid == 0)
        def _():
            pltpu.sync_copy(lin_hbm.at[pl.ds(ebase, _NCH0 * _CH)], idx_v)
            pltpu.sync_copy(w_hbm.at[pl.ds(nbase, _NCH0 * _CNODES)], wbuf)

        @pl.when(cid != 0)
        def _():
            pltpu.sync_copy(lin_hbm.at[pl.ds(ebase, _NCH1 * _CH)],
                            idx_v.at[pl.ds(0, _NCH1 * _CH)])
            pltpu.sync_copy(w_hbm.at[pl.ds(nbase, _NCH1 * _CNODES)],
                            wbuf.at[pl.ds(0, _NCH1 * _CNODES)])

        def src(c):
            return vt_hbm.at[idx_v.at[pl.ds(c * _CH, _CH)]]

        def adst(c):
            return agg_hbm.at[pl.ds(nbase + c * _CNODES, _CNODES)]

        def compute(c, buf, abuf):
            for g in range(_CNODES):
                nl = c * _CNODES + g
                w16 = wbuf[nl]

                def edge_body(d, accs):
                    e = g * DEG + d
                    ws = _perm16(w16, jnp.full((16,), d, jnp.int32))
                    return tuple(
                        a + ws * buf[e, pl.ds(cc * 16, 16)]
                        for cc, a in enumerate(accs))

                accs = lax.fori_loop(
                    0, DEG, edge_body,
                    tuple(jnp.zeros((16,), jnp.float32) for _ in range(16)))
                for cc in range(16):
                    abuf[g, pl.ds(cc * 16, 16)] = accs[cc]

        pltpu.async_copy(src(0), buf0, sem0)
        pltpu.async_copy(src(1), buf1, sem1)

        def do(c, buf, abuf, sem, semw, more):
            pltpu.make_async_copy(src(c), buf, sem).wait()

            @pl.when(c >= 2)
            def _():
                pltpu.make_async_copy(abuf, adst(c - 2), semw).wait()

            compute(c, buf, abuf)
            pltpu.async_copy(abuf, adst(c), semw)

            @pl.when(more)
            def _():
                pltpu.async_copy(src(c + 2), buf, sem)

        def step(p, carry):
            do(p * 2, buf0, abuf0, sem0, semw0, p * 2 + 2 < nch)
            do(p * 2 + 1, buf1, abuf1, sem1, semw1, p * 2 + 3 < nch)
            return carry

        lax.fori_loop(0, nch // 2, step, 0)
        pltpu.make_async_copy(abuf0, adst(nch - 2), semw0).wait()
        pltpu.make_async_copy(abuf1, adst(nch - 1), semw1).wait()

    return agg(lin_pad, w_pad, vt_flat)


def _out_body(agg_ref, ns_ref, fcw_ref, fcb_ref, out_ref):
    fc = lax.dot_general(agg_ref[...], fcw_ref[...],
                         (((1,), (1,)), ((), ())),
                         preferred_element_type=jnp.float32) + fcb_ref[...]
    out_ref[...] = ns_ref[...] + jnp.maximum(fc, 0.0)


def _output(agg_pad, ns, fc_w, fc_b):
    grid = (N // SBLK,)
    return pl.pallas_call(
        _out_body,
        grid=grid,
        in_specs=[
            pl.BlockSpec((SBLK, U), lambda i: (i, 0)),
            pl.BlockSpec((SBLK, U), lambda i: (i, 0)),
            pl.BlockSpec((U, U), lambda i: (0, 0)),
            pl.BlockSpec((1, U), lambda i: (0, 0)),
        ],
        out_specs=pl.BlockSpec((SBLK, U), lambda i: (i, 0)),
        out_shape=jax.ShapeDtypeStruct((N, U), jnp.float32),
    )(agg_pad, ns, fc_w, fc_b)


def kernel(node_state, adjacency, point_enc, relation_enc, point_enc_w,
           relation_enc_w, q_w, k_w, v_w, fc_w, fc_b):
    pe2 = point_enc.reshape(N, 1)
    ns, qs, lin, kt, vt = _projtab(node_state, pe2, adjacency, relation_enc,
                                   point_enc_w, q_w, k_w, v_w,
                                   relation_enc_w)
    lin_pad = jnp.pad(lin.reshape(EDGES), (0, EPAD - EDGES))
    qs_pad = jnp.pad(qs, ((0, NPAD - N), (0, 0)))
    raw_pad = _sc_scores(lin_pad, qs_pad, kt.reshape(RELS * N, U))
    rel_pad = jnp.pad(relation_enc, ((0, NPAD - N), (0, 0)))
    adj_pad = jnp.pad(adjacency, ((0, NPAD - N), (0, 0)))
    w_pad = _softmax0(raw_pad, rel_pad, adj_pad)
    agg_pad = _sc_agg(lin_pad, w_pad, vt.reshape(RELS * N, U))
    out = _output(agg_pad, ns, fc_w, fc_b.reshape(1, U))
    return out


# speedup vs baseline: 1.0003x; 1.0003x over previous
import functools

import jax
import jax.numpy as jnp
from jax import lax
from jax.experimental import pallas as pl
from jax.experimental.pallas import tpu as pltpu
from jax.experimental.pallas import tpu_sc as plsc

N = 10000
DEG = 16
U = 256
RELS = 8
ENTS = 4
EDGES = N * DEG

NBLK = 1000
SBLK = 200
NEG = -1000000000.0

_NW = 32
NPAD = 10240
EPAD = NPAD * DEG
_CH = 64
_CNODES = _CH // DEG
_NCHP = EPAD // _CH // 16
_NCH0 = 112
_NCH1 = 48

def _projtab_body(ns_ref, pe_ref, adj_ref, rel_ref, pw_ref, qw_ref, kw_ref,
                  vw_ref, relw_ref, nso_ref, qso_ref, lin_ref, kt_ref,
                  vt_ref, nsb, kws, vws):
    r = pl.program_id(1)

    @pl.when(jnp.logical_and(pl.program_id(0) == 0, r == 0))
    def _():
        for rr in range(RELS):
            kws[rr] = jnp.dot(relw_ref[rr], kw_ref[...],
                              preferred_element_type=jnp.float32
                              ).astype(jnp.bfloat16)
            vws[rr] = jnp.dot(relw_ref[rr], vw_ref[...],
                              preferred_element_type=jnp.float32
                              ).astype(jnp.bfloat16)

    @pl.when(r == 0)
    def _():
        x = ns_ref[...]
        pe = pe_ref[...]
        acc = jnp.zeros((NBLK, U), jnp.float32)
        for e in range(ENTS):
            pm = jnp.where(pe == e, 1.0, 0.0)
            acc = acc + pm * jnp.dot(x, pw_ref[e],
                                     preferred_element_type=jnp.float32)
        nsb[...] = acc.astype(jnp.bfloat16)
        nso_ref[...] = acc
        qso_ref[...] = jnp.dot(acc, qw_ref[...],
                               preferred_element_type=jnp.float32) * (1.0 / U)
        lin_ref[...] = rel_ref[...] * N + jnp.maximum(adj_ref[...] - 1, 0)

    x = nsb[...]
    kt_ref[0] = jnp.dot(x, kws[r], preferred_element_type=jnp.float32)
    vt_ref[0] = jnp.dot(x, vws[r], preferred_element_type=jnp.float32)


def _projtab(node_state, point_enc, adjacency, relation_enc, pw, qw, kw, vw,
             relw):
    grid = (N // NBLK, RELS)
    return pl.pallas_call(
        _projtab_body,
        grid=grid,
        in_specs=[
            pl.BlockSpec((NBLK, U), lambda i, r: (i, 0)),
            pl.BlockSpec((NBLK, 1), lambda i, r: (i, 0)),
            pl.BlockSpec((NBLK, DEG), lambda i, r: (i, 0)),
            pl.BlockSpec((NBLK, DEG), lambda i, r: (i, 0)),
            pl.BlockSpec((ENTS, U, U), lambda i, r: (0, 0, 0)),
            pl.BlockSpec((U, U), lambda i, r: (0, 0)),
            pl.BlockSpec((U, U), lambda i, r: (0, 0)),
            pl.BlockSpec((U, U), lambda i, r: (0, 0)),
            pl.BlockSpec((RELS, U, U), lambda i, r: (0, 0, 0)),
        ],
        out_specs=[
            pl.BlockSpec((NBLK, U), lambda i, r: (i, 0)),
            pl.BlockSpec((NBLK, U), lambda i, r: (i, 0)),
            pl.BlockSpec((NBLK, DEG), lambda i, r: (i, 0)),
            pl.BlockSpec((1, NBLK, U), lambda i, r: (r, i, 0)),
            pl.BlockSpec((1, NBLK, U), lambda i, r: (r, i, 0)),
        ],
        out_shape=[
            jax.ShapeDtypeStruct((N, U), jnp.float32),
            jax.ShapeDtypeStruct((N, U), jnp.float32),
            jax.ShapeDtypeStruct((N, DEG), jnp.int32),
            jax.ShapeDtypeStruct((RELS, N, U), jnp.float32),
            jax.ShapeDtypeStruct((RELS, N, U), jnp.float32),
        ],
        scratch_shapes=[
            pltpu.VMEM((NBLK, U), jnp.bfloat16),
            pltpu.VMEM((RELS, U, U), jnp.bfloat16),
            pltpu.VMEM((RELS, U, U), jnp.bfloat16),
        ],
    )(node_state, point_enc, adjacency, relation_enc, pw, qw, kw, vw, relw)


def _perm16(x, idx):
    return lax.gather(
        x, idx[:, None],
        lax.GatherDimensionNumbers(offset_dims=(), collapsed_slice_dims=(0,),
                                   start_index_map=(0,)),
        slice_sizes=(1,),
        mode=lax.GatherScatterMode.PROMISE_IN_BOUNDS)


def _lanesum(x):
    lanes = lax.iota(jnp.int32, 16)
    for sh in (1, 2, 4, 8):
        x = x + _perm16(x, lanes ^ sh)
    return x


def _sc_scores(lin_pad, qs_pad, kt_flat):
    mesh = plsc.VectorSubcoreMesh(core_axis_name="c", subcore_axis_name="s")

    @functools.partial(
        pl.kernel,
        mesh=mesh,
        out_type=jax.ShapeDtypeStruct((NPAD, DEG), jnp.float32),
        scratch_types=[
            pltpu.VMEM((_NCH0 * _CH,), jnp.int32),
            pltpu.VMEM((_NCH0 * _CNODES, DEG), jnp.float32),
            pltpu.VMEM((_CH, U), jnp.float32),
            pltpu.VMEM((_CH, U), jnp.float32),
            pltpu.VMEM((_CNODES, U), jnp.float32),
            pltpu.VMEM((_CNODES, U), jnp.float32),
            pltpu.SemaphoreType.DMA,
            pltpu.SemaphoreType.DMA,
            pltpu.SemaphoreType.DMA,
            pltpu.SemaphoreType.DMA,
        ],
    )
    def scores(lin_hbm, qs_hbm, kt_hbm, raw_hbm, idx_v, sbuf, buf0, buf1,
               qb0, qb1, sem0, sem1, semq0, semq1):
        cid = lax.axis_index("c")
        sid = lax.axis_index("s")
        cbase = sid * _NCHP + cid * _NCH0
        nch = jnp.where(cid == 0, _NCH0, _NCH1)
        ebase = cbase * _CH
        nbase = cbase * _CNODES

        @pl.when(cid == 0)
        def _():
            pltpu.sync_copy(lin_hbm.at[pl.ds(ebase, _NCH0 * _CH)], idx_v)

        @pl.when(cid != 0)
        def _():
            pltpu.sync_copy(lin_hbm.at[pl.ds(ebase, _NCH1 * _CH)],
                            idx_v.at[pl.ds(0, _NCH1 * _CH)])

        def src(c):
            return kt_hbm.at[idx_v.at[pl.ds(c * _CH, _CH)]]

        def qsrc(c):
            return qs_hbm.at[pl.ds(nbase + c * _CNODES, _CNODES)]

        def compute(c, buf, qbuf):
            for g in range(_CNODES):
                nl = c * _CNODES + g
                qv = tuple(qbuf[g, pl.ds(cc * 16, 16)] for cc in range(16))

                def edge_body(d, srow):
                    e = g * DEG + d
                    acc = qv[0] * buf[e, pl.ds(0, 16)]
                    for cc in range(1, 16):
                        acc = acc + qv[cc] * buf[e, pl.ds(cc * 16, 16)]
                    s = _lanesum(acc)
                    return jnp.where(
                        lax.iota(jnp.int32, 16) == d, s, srow)

                srow = lax.fori_loop(0, DEG, edge_body,
                                     jnp.zeros((16,), jnp.float32))
                sbuf[nl] = srow

        pltpu.async_copy(src(0), buf0, sem0)
        pltpu.async_copy(qsrc(0), qb0, semq0)
        pltpu.async_copy(src(1), buf1, sem1)
        pltpu.async_copy(qsrc(1), qb1, semq1)

        def do(c, buf, qbuf, sem, semq, more):
            pltpu.make_async_copy(src(c), buf, sem).wait()
            pltpu.make_async_copy(qsrc(c), qbuf, semq).wait()
            compute(c, buf, qbuf)

            @pl.when(more)
            def _():
                pltpu.async_copy(src(c + 2), buf, sem)
                pltpu.async_copy(qsrc(c + 2), qbuf, semq)

        def step(p, carry):
            do(p * 2, buf0, qb0, sem0, semq0, p * 2 + 2 < nch)
            do(p * 2 + 1, buf1, qb1, sem1, semq1, p * 2 + 3 < nch)
            return carry

        lax.fori_loop(0, nch // 2, step, 0)

        @pl.when(cid == 0)
        def _():
            pltpu.sync_copy(sbuf,
                            raw_hbm.at[pl.ds(nbase, _NCH0 * _CNODES)])

        @pl.when(cid != 0)
        def _():
            pltpu.sync_copy(sbuf.at[pl.ds(0, _NCH1 * _CNODES)],
                            raw_hbm.at[pl.ds(nbase, _NCH1 * _CNODES)])

    return scores(lin_pad, qs_pad, kt_flat)


def _softmax_body(raw_ref, rel_ref, adj_ref, w_ref):
    raw = raw_ref[...]
    raw = jnp.where(adj_ref[...] == 0, 0.0, raw)
    raw = jnp.where(rel_ref[...] == 0, NEG, raw)
    m = jnp.max(raw, axis=0, keepdims=True)
    e = jnp.exp(raw - m)
    s = jnp.sum(e, axis=0, keepdims=True)
    w = e / s
    w_ref[...] = jnp.where(adj_ref[...] == 0, 0.0, w)


def _softmax0(raw_pad, rel_pad, adj_pad):
    return pl.pallas_call(
        _softmax_body,
        out_shape=jax.ShapeDtypeStruct((NPAD, DEG), jnp.float32),
    )(raw_pad, rel_pad, adj_pad)


def _sc_agg(lin_pad, w_pad, vt_flat):
    mesh = plsc.VectorSubcoreMesh(core_axis_name="c", subcore_axis_name="s")

    @functools.partial(
        pl.kernel,
        mesh=mesh,
        out_type=jax.ShapeDtypeStruct((NPAD, U), jnp.float32),
        scratch_types=[
            pltpu.VMEM((_NCH0 * _CH,), jnp.int32),
            pltpu.VMEM((_NCH0 * _CNODES, DEG), jnp.float32),
            pltpu.VMEM((_CNODES, U), jnp.float32),
            pltpu.VMEM((_CH, U), jnp.float32),
            pltpu.VMEM((_CH, U), jnp.float32),
            pltpu.SemaphoreType.DMA,
            pltpu.SemaphoreType.DMA,
        ],
    )
    def agg(lin_hbm, w_hbm, vt_hbm, agg_hbm, idx_v, wbuf, abuf, buf0, buf1,
            sem0, sem1):
        cid = lax.axis_index("c")
        sid = lax.axis_index("s")
        cbase = sid * _NCHP + cid * _NCH0
        nch = jnp.where(cid == 0, _NCH0, _NCH1)
        ebase = cbase * _CH
        nbase = cbase * _CNODES

        @pl.when(cid == 0)
        def _():
            pltpu.sync_copy(lin_hbm.at[pl.ds(ebase, _NCH0 * _CH)], idx_v)
            pltpu.sync_copy(w_hbm.at[pl.ds(nbase, _NCH0 * _CNODES)], wbuf)

        @pl.when(cid != 0)
        def _():
            pltpu.sync_copy(lin_hbm.at[pl.ds(ebase, _NCH1 * _CH)],
                            idx_v.at[pl.ds(0, _NCH1 * _CH)])
            pltpu.sync_copy(w_hbm.at[pl.ds(nbase, _NCH1 * _CNODES)],
                            wbuf.at[pl.ds(0, _NCH1 * _CNODES)])

        def src(c):
            return vt_hbm.at[idx_v.at[pl.ds(c * _CH, _CH)]]

        def compute(c, buf):
            for g in range(_CNODES):
                nl = c * _CNODES + g
                w16 = wbuf[nl]

                def edge_body(d, accs):
                    e = g * DEG + d
                    ws = _perm16(w16, jnp.full((16,), d, jnp.int32))
                    return tuple(
                        a + ws * buf[e, pl.ds(cc * 16, 16)]
                        for cc, a in enumerate(accs))

                accs = lax.fori_loop(
                    0, DEG, edge_body,
                    tuple(jnp.zeros((16,), jnp.float32) for _ in range(16)))
                for cc in range(16):
                    abuf[g, pl.ds(cc * 16, 16)] = accs[cc]
            pltpu.sync_copy(
                abuf,
                agg_hbm.at[pl.ds(nbase + c * _CNODES, _CNODES)])

        pltpu.async_copy(src(0), buf0, sem0)
        pltpu.async_copy(src(1), buf1, sem1)

        def do(c, buf, sem, more):
            pltpu.make_async_copy(src(c), buf, sem).wait()
            compute(c, buf)

            @pl.when(more)
            def _():
                pltpu.async_copy(src(c + 2), buf, sem)

        def step(p, carry):
            do(p * 2, buf0, sem0, p * 2 + 2 < nch)
            do(p * 2 + 1, buf1, sem1, p * 2 + 3 < nch)
            return carry

        lax.fori_loop(0, nch // 2, step, 0)

    return agg(lin_pad, w_pad, vt_flat)


def _out_body(agg_ref, ns_ref, fcw_ref, fcb_ref, out_ref):
    fc = lax.dot_general(agg_ref[...], fcw_ref[...],
                         (((1,), (1,)), ((), ())),
                         preferred_element_type=jnp.float32) + fcb_ref[...]
    out_ref[...] = ns_ref[...] + jnp.maximum(fc, 0.0)


def _output(agg_pad, ns, fc_w, fc_b):
    grid = (N // SBLK,)
    return pl.pallas_call(
        _out_body,
        grid=grid,
        in_specs=[
            pl.BlockSpec((SBLK, U), lambda i: (i, 0)),
            pl.BlockSpec((SBLK, U), lambda i: (i, 0)),
            pl.BlockSpec((U, U), lambda i: (0, 0)),
            pl.BlockSpec((1, U), lambda i: (0, 0)),
        ],
        out_specs=pl.BlockSpec((SBLK, U), lambda i: (i, 0)),
        out_shape=jax.ShapeDtypeStruct((N, U), jnp.float32),
    )(agg_pad, ns, fc_w, fc_b)


def kernel(node_state, adjacency, point_enc, relation_enc, point_enc_w,
           relation_enc_w, q_w, k_w, v_w, fc_w, fc_b):
    pe2 = point_enc.reshape(N, 1)
    ns, qs, lin, kt, vt = _projtab(node_state, pe2, adjacency, relation_enc,
                                   point_enc_w, q_w, k_w, v_w,
                                   relation_enc_w)
    lin_pad = jnp.pad(lin.reshape(EDGES), (0, EPAD - EDGES))
    qs_pad = jnp.pad(qs, ((0, NPAD - N), (0, 0)))
    raw_pad = _sc_scores(lin_pad, qs_pad, kt.reshape(RELS * N, U))
    rel_pad = jnp.pad(relation_enc, ((0, NPAD - N), (0, 0)))
    adj_pad = jnp.pad(adjacency, ((0, NPAD - N), (0, 0)))
    w_pad = _softmax0(raw_pad, rel_pad, adj_pad)
    agg_pad = _sc_agg(lin_pad, w_pad, vt.reshape(RELS * N, U))
    out = _output(agg_pad, ns, fc_w, fc_b.reshape(1, U))
    return out
